# baseline (speedup 1.0000x reference)
import functools

import jax
import jax.numpy as jnp
from jax import lax
from jax.experimental import pallas as pl
from jax.experimental.pallas import tpu as pltpu
from jax.experimental.pallas import tpu_sc as plsc

E = 8
D = 768
T = 8192
A = 2 * T
M = 256
NSLOT = A + E * M
NB = NSLOT // M
NBP = 80

NC = 2
NS = 16
NW = NC * NS
APW = A // NW
TPW = T // NW

RBLK = 512
_NEG_INF = -1e30


def _router_body(x_ref, wr_ref, eid_ref, wgt_ref, cnt_ref):
    xb = x_ref[...]
    logits = jnp.dot(xb, wr_ref[...])
    m1 = jnp.max(logits, axis=-1)
    i1 = jnp.argmax(logits, axis=-1)
    cols = lax.broadcasted_iota(jnp.int32, logits.shape, 1)
    masked = jnp.where(cols == i1[:, None], _NEG_INF, logits)
    m2 = jnp.max(masked, axis=-1)
    i2 = jnp.argmax(masked, axis=-1)
    w1 = 1.0 / (1.0 + jnp.exp(m2 - m1))
    eid_ref[0:1, :] = i1[None, :]
    eid_ref[1:2, :] = i2[None, :]
    wgt_ref[0:1, :] = w1[None, :]
    wgt_ref[1:2, :] = (1.0 - w1)[None, :]
    lanes = lax.broadcasted_iota(jnp.int32, (RBLK, 128), 1)
    cnt_ref[0, 0] = jnp.sum((i1[:, None] == lanes).astype(jnp.int32), axis=0)[None, :]
    cnt_ref[1, 0] = jnp.sum((i2[:, None] == lanes).astype(jnp.int32), axis=0)[None, :]


def _router(xf, Wr):
    nb = T // RBLK
    return pl.pallas_call(
        _router_body,
        grid=(nb,),
        in_specs=[
            pl.BlockSpec((RBLK, D), lambda b: (b, 0)),
            pl.BlockSpec((D, E), lambda b: (0, 0)),
        ],
        out_specs=[
            pl.BlockSpec((2, RBLK), lambda b: (0, b)),
            pl.BlockSpec((2, RBLK), lambda b: (0, b)),
            pl.BlockSpec((2, 1, 1, 128), lambda b: (0, b, 0, 0)),
        ],
        out_shape=[
            jax.ShapeDtypeStruct((2, T), jnp.int32),
            jax.ShapeDtypeStruct((2, T), jnp.float32),
            jax.ShapeDtypeStruct((2, nb, 1, 128), jnp.int32),
        ],
        compiler_params=pltpu.CompilerParams(
            dimension_semantics=("parallel",),
        ),
    )(xf, Wr)


def _iota16():
    return lax.iota(jnp.int32, 16)


def _lane(v, e):
    return lax.squeeze(lax.slice(v, (e,), (e + 1,)), (0,))


def _dispatch_body(eid_hbm, wgt_hbm, cnt_hbm, x_hbm,
                  pos_hbm, xs_hbm, wsl_hbm, bexp_hbm,
                  eid_v, wgt_v, slots_v, tok_v, cnt_v, rows_v, bexp_v, sem):
    cid = lax.axis_index("c")
    sid = lax.axis_index("s")
    wid = sid * NC + cid

    pltpu.sync_copy(eid_hbm.at[pl.ds(wid * 4, 4)], eid_v)
    pltpu.sync_copy(wgt_hbm.at[pl.ds(wid * 4, 4)], wgt_v)
    pltpu.sync_copy(cnt_hbm, cnt_v)

    zero = jnp.zeros((16,), jnp.int32)
    tot = zero
    pre = zero
    wid_v = jnp.full((16,), wid, jnp.int32)
    for r in range(NW):
        row = cnt_v[r, pl.ds(0, 16)]
        tot = tot + row
        r_v = jnp.full((16,), r, jnp.int32)
        pre = pre + jnp.where(r_v < wid_v, row, zero)
    aligned = ((tot + (M - 1)) >> 8) << 8
    base = plsc.cumsum(aligned) - aligned
    start = base + pre

    starts = [_lane(start, e) for e in range(E)]
    for j in range(APW // 16):
        r, o = j // 8, (j % 8) * 16
        v = eid_v[r, pl.ds(o, 16)]
        slot = zero
        for e in range(E):
            m = v == e
            c = plsc.cumsum(m.astype(jnp.int32))
            slot = jnp.where(m, starts[e] + c - 1, slot)
            starts[e] = starts[e] + jnp.max(c)
        slots_v[r, pl.ds(o, 16)] = slot
        flat = wid * APW + j * 16
        tok_v[r, pl.ds(o, 16)] = (flat + _iota16()) & (T - 1)

    pltpu.sync_copy(slots_v, pos_hbm.at[pl.ds(wid * 4, 4)])

    for c in range(4):
        pltpu.async_copy(x_hbm.at[tok_v.at[c]], rows_v, sem).wait()
        pltpu.sync_copy(rows_v, xs_hbm.at[slots_v.at[c]])
        pltpu.sync_copy(wgt_v.at[c], wsl_hbm.at[slots_v.at[c]])

    @pl.when(wid == 0)
    def _():
        for q in range(NBP // 16):
            bstart = (_iota16() + q * 16) * M
            acc = jnp.zeros((16,), jnp.int32)
            for e in range(1, E):
                acc = acc + (bstart >= _lane(base, e)).astype(jnp.int32)
            bexp_v[pl.ds(q * 16, 16)] = acc
        pltpu.sync_copy(bexp_v, bexp_hbm)


def _dispatch(eid2, wgt2, cnt2, xf):
    mesh = plsc.VectorSubcoreMesh(core_axis_name="c", subcore_axis_name="s", num_cores=NC, num_subcores=NS)
    f = pl.kernel(
        _dispatch_body,
        out_type=[
            jax.ShapeDtypeStruct((A // 128, 128), jnp.int32),
            jax.ShapeDtypeStruct((NSLOT, D), jnp.float32),
            jax.ShapeDtypeStruct((NSLOT,), jnp.float32),
            jax.ShapeDtypeStruct((NBP,), jnp.int32),
        ],
        mesh=mesh,
        scratch_types=[
            pltpu.VMEM((4, 128), jnp.int32),
            pltpu.VMEM((4, 128), jnp.float32),
            pltpu.VMEM((4, 128), jnp.int32),
            pltpu.VMEM((4, 128), jnp.int32),
            pltpu.VMEM((NW, 128), jnp.int32),
            pltpu.VMEM((128, D), jnp.float32),
            pltpu.VMEM((NBP,), jnp.int32),
            pltpu.SemaphoreType.DMA,
        ],
        compiler_params=pltpu.CompilerParams(needs_layout_passes=False),
    )
    return f(eid2, wgt2, cnt2, xf)


def _expert_body(bexp_ref, xs_ref, we_ref, be_ref, wsl_ref, ys_ref):
    xb = xs_ref[...].astype(jnp.bfloat16)
    h = jnp.dot(xb, we_ref[0], preferred_element_type=jnp.float32) + be_ref[0]
    h = 0.5 * h * (1.0 + lax.erf(h * 0.7071067811865476))
    ys_ref[...] = wsl_ref[0, 0][:, None] * h


def _experts(bexp, xs, We_bf, be3, wsl3):
    grid_spec = pltpu.PrefetchScalarGridSpec(
        num_scalar_prefetch=1,
        grid=(NB,),
        in_specs=[
            pl.BlockSpec((M, D), lambda b, bm: (b, 0)),
            pl.BlockSpec((1, D, D), lambda b, bm: (bm[b], 0, 0)),
            pl.BlockSpec((1, 1, D), lambda b, bm: (bm[b], 0, 0)),
            pl.BlockSpec((1, 1, M), lambda b, bm: (b, 0, 0)),
        ],
        out_specs=pl.BlockSpec((M, D), lambda b, bm: (b, 0)),
    )
    return pl.pallas_call(
        _expert_body,
        grid_spec=grid_spec,
        out_shape=jax.ShapeDtypeStruct((NSLOT, D), jnp.float32),
        compiler_params=pltpu.CompilerParams(
            dimension_semantics=("arbitrary",),
        ),
    )(bexp, xs, We_bf, be3, wsl3)


def _combine_body(ys_hbm, pos_hbm, ya_hbm, yb_hbm, idx_v, rows_v, sem):
    cid = lax.axis_index("c")
    sid = lax.axis_index("s")
    wid = sid * NC + cid

    for h in range(2):
        row0 = wid * 2 + h
        dst = pl.ds(wid * TPW + h * 128, 128)
        pltpu.sync_copy(pos_hbm.at[row0], idx_v)
        pltpu.async_copy(ys_hbm.at[idx_v], rows_v, sem).wait()
        pltpu.sync_copy(rows_v, ya_hbm.at[dst])
        pltpu.sync_copy(pos_hbm.at[64 + row0], idx_v)
        pltpu.async_copy(ys_hbm.at[idx_v], rows_v, sem).wait()
        pltpu.sync_copy(rows_v, yb_hbm.at[dst])


def _combine(ys, pos2):
    mesh = plsc.VectorSubcoreMesh(core_axis_name="c", subcore_axis_name="s", num_cores=NC, num_subcores=NS)
    f = pl.kernel(
        _combine_body,
        out_type=[
            jax.ShapeDtypeStruct((T, D), jnp.float32),
            jax.ShapeDtypeStruct((T, D), jnp.float32),
        ],
        mesh=mesh,
        scratch_types=[
            pltpu.VMEM((128,), jnp.int32),
            pltpu.VMEM((128, D), jnp.float32),
            pltpu.SemaphoreType.DMA,
        ],
        compiler_params=pltpu.CompilerParams(needs_layout_passes=False),
    )
    return f(ys, pos2)


def _sum_body(a_ref, b_ref, o_ref):
    o_ref[...] = a_ref[...] + b_ref[...]


def _sum(ya, yb):
    return pl.pallas_call(
        _sum_body,
        grid=(T // RBLK,),
        in_specs=[
            pl.BlockSpec((RBLK, D), lambda b: (b, 0)),
            pl.BlockSpec((RBLK, D), lambda b: (b, 0)),
        ],
        out_specs=pl.BlockSpec((RBLK, D), lambda b: (b, 0)),
        out_shape=jax.ShapeDtypeStruct((T, D), jnp.float32),
        compiler_params=pltpu.CompilerParams(
            dimension_semantics=("parallel",),
        ),
    )(ya, yb)


@jax.jit
def kernel(x, Wr, We, be):
    Bx, Sx, Dx = x.shape
    xf = x.reshape(T, D)

    eid, wgt, cnt = _router(xf, Wr)
    eid2 = eid.reshape(A // 128, 128)
    wgt2 = wgt.reshape(A // 128, 128)
    cnt2 = cnt.reshape(NW, 128)

    pos2, xs, wsl, bexp = _dispatch(eid2, wgt2, cnt2, xf)

    We_bf = We.astype(jnp.bfloat16)
    be3 = be.reshape(E, 1, D)
    wsl3 = wsl.reshape(NB, 1, M)
    ys = _experts(bexp, xs, We_bf, be3, wsl3)

    ya, yb = _combine(ys, pos2)
    out = _sum(ya, yb)
    return out.reshape(Bx, Sx, Dx)


# speedup vs baseline: 1.1694x; 1.1694x over previous
import functools

import jax
import jax.numpy as jnp
from jax import lax
from jax.experimental import pallas as pl
from jax.experimental.pallas import tpu as pltpu
from jax.experimental.pallas import tpu_sc as plsc

E = 8
D = 768
T = 8192
A = 2 * T
M = 256
NSLOT = A + E * M
NB = NSLOT // M
NBP = 80

NC = 2
NS = 16
NW = NC * NS
APW = A // NW
TPW = T // NW

RBLK = 512
_NEG_INF = -1e30


def _pack_bf16(h):
    u = lax.bitcast_convert_type(h, jnp.uint32)
    r = (u + 0x7FFF + ((u >> 16) & 1)) >> 16
    lo = r[:, : D // 2] & 0xFFFF
    hi = r[:, D // 2 :] & 0xFFFF
    return lax.bitcast_convert_type(lo | (hi << 16), jnp.int32)


def _unpack_bf16(p):
    v = lax.bitcast_convert_type(p, jnp.uint32)
    lo = (v & 0xFFFF).astype(jnp.uint16)
    hi = (v >> 16).astype(jnp.uint16)
    return jnp.concatenate(
        [
            lax.bitcast_convert_type(lo, jnp.bfloat16),
            lax.bitcast_convert_type(hi, jnp.bfloat16),
        ],
        axis=1,
    )


def _router_body(x_ref, wr_ref, eid_ref, wgt_ref, cnt_ref, xp_ref):
    xb = x_ref[...]
    xp_ref[...] = _pack_bf16(xb)
    logits = jnp.dot(xb, wr_ref[...])
    m1 = jnp.max(logits, axis=-1)
    i1 = jnp.argmax(logits, axis=-1)
    cols = lax.broadcasted_iota(jnp.int32, logits.shape, 1)
    masked = jnp.where(cols == i1[:, None], _NEG_INF, logits)
    m2 = jnp.max(masked, axis=-1)
    i2 = jnp.argmax(masked, axis=-1)
    w1 = 1.0 / (1.0 + jnp.exp(m2 - m1))
    eid_ref[0:1, :] = i1[None, :]
    eid_ref[1:2, :] = i2[None, :]
    wgt_ref[0:1, :] = w1[None, :]
    wgt_ref[1:2, :] = (1.0 - w1)[None, :]
    lanes = lax.broadcasted_iota(jnp.int32, (RBLK, 128), 1)
    cnt_ref[0, 0] = jnp.sum((i1[:, None] == lanes).astype(jnp.int32), axis=0)[None, :]
    cnt_ref[1, 0] = jnp.sum((i2[:, None] == lanes).astype(jnp.int32), axis=0)[None, :]


def _router(xf, Wr):
    nb = T // RBLK
    return pl.pallas_call(
        _router_body,
        grid=(nb,),
        in_specs=[
            pl.BlockSpec((RBLK, D), lambda b: (b, 0)),
            pl.BlockSpec((D, E), lambda b: (0, 0)),
        ],
        out_specs=[
            pl.BlockSpec((2, RBLK), lambda b: (0, b)),
            pl.BlockSpec((2, RBLK), lambda b: (0, b)),
            pl.BlockSpec((2, 1, 1, 128), lambda b: (0, b, 0, 0)),
            pl.BlockSpec((RBLK, D // 2), lambda b: (b, 0)),
        ],
        out_shape=[
            jax.ShapeDtypeStruct((2, T), jnp.int32),
            jax.ShapeDtypeStruct((2, T), jnp.float32),
            jax.ShapeDtypeStruct((2, nb, 1, 128), jnp.int32),
            jax.ShapeDtypeStruct((T, D // 2), jnp.int32),
        ],
        compiler_params=pltpu.CompilerParams(
            dimension_semantics=("parallel",),
        ),
    )(xf, Wr)


def _iota16():
    return lax.iota(jnp.int32, 16)


def _lane(v, e):
    return lax.squeeze(lax.slice(v, (e,), (e + 1,)), (0,))


def _dispatch_body(eid_hbm, wgt_hbm, cnt_hbm, xp_hbm,
                  pos_hbm, xs_hbm, wsl_hbm, bexp_hbm,
                  eid_v, wgt_v, slots_v, tok_v, cnt_v, rows0_v, rows1_v, bexp_v,
                  sg0, sg1, ss0, ss1):
    cid = lax.axis_index("c")
    sid = lax.axis_index("s")
    wid = sid * NC + cid

    pltpu.sync_copy(eid_hbm.at[pl.ds(wid * 4, 4)], eid_v)
    pltpu.sync_copy(wgt_hbm.at[pl.ds(wid * 4, 4)], wgt_v)
    pltpu.sync_copy(cnt_hbm, cnt_v)

    for j in range(APW // 16):
        r, o = j // 8, (j % 8) * 16
        flat = wid * APW + j * 16
        tok_v[r, pl.ds(o, 16)] = (flat + _iota16()) & (T - 1)
    g0 = pltpu.async_copy(xp_hbm.at[tok_v.at[0]], rows0_v, sg0)
    g1 = pltpu.async_copy(xp_hbm.at[tok_v.at[1]], rows1_v, sg1)

    zero = jnp.zeros((16,), jnp.int32)
    tot = zero
    pre = zero
    wid_v = jnp.full((16,), wid, jnp.int32)
    for r in range(NW):
        row = cnt_v[r, pl.ds(0, 16)]
        tot = tot + row
        r_v = jnp.full((16,), r, jnp.int32)
        pre = pre + jnp.where(r_v < wid_v, row, zero)
    aligned = ((tot + (M - 1)) >> 8) << 8
    base = plsc.cumsum(aligned) - aligned
    start = base + pre

    starts = [_lane(start, e) for e in range(E)]
    for j in range(APW // 16):
        r, o = j // 8, (j % 8) * 16
        v = eid_v[r, pl.ds(o, 16)]
        slot = zero
        for e in range(E):
            m = v == e
            c = plsc.cumsum(m.astype(jnp.int32))
            slot = jnp.where(m, starts[e] + c - 1, slot)
            starts[e] = starts[e] + jnp.max(c)
        slots_v[r, pl.ds(o, 16)] = slot

    pltpu.sync_copy(slots_v, pos_hbm.at[pl.ds(wid * 4, 4)])

    g0.wait()
    s0 = pltpu.async_copy(rows0_v, xs_hbm.at[slots_v.at[0]], ss0)
    g1.wait()
    s1 = pltpu.async_copy(rows1_v, xs_hbm.at[slots_v.at[1]], ss1)
    s0.wait()
    g2 = pltpu.async_copy(xp_hbm.at[tok_v.at[2]], rows0_v, sg0)
    s1.wait()
    g3 = pltpu.async_copy(xp_hbm.at[tok_v.at[3]], rows1_v, sg1)
    g2.wait()
    s2 = pltpu.async_copy(rows0_v, xs_hbm.at[slots_v.at[2]], ss0)
    g3.wait()
    s3 = pltpu.async_copy(rows1_v, xs_hbm.at[slots_v.at[3]], ss1)
    for c in range(4):
        pltpu.sync_copy(wgt_v.at[c], wsl_hbm.at[slots_v.at[c]])
    s2.wait()
    s3.wait()

    @pl.when(wid == 0)
    def _():
        for q in range(NBP // 16):
            bstart = (_iota16() + q * 16) * M
            acc = jnp.zeros((16,), jnp.int32)
            for e in range(1, E):
                acc = acc + (bstart >= _lane(base, e)).astype(jnp.int32)
            bexp_v[pl.ds(q * 16, 16)] = acc
        pltpu.sync_copy(bexp_v, bexp_hbm)


def _dispatch(eid2, wgt2, cnt2, xp):
    mesh = plsc.VectorSubcoreMesh(core_axis_name="c", subcore_axis_name="s", num_cores=NC, num_subcores=NS)
    f = pl.kernel(
        _dispatch_body,
        out_type=[
            jax.ShapeDtypeStruct((A // 128, 128), jnp.int32),
            jax.ShapeDtypeStruct((NSLOT, D // 2), jnp.int32),
            jax.ShapeDtypeStruct((NSLOT,), jnp.float32),
            jax.ShapeDtypeStruct((NBP,), jnp.int32),
        ],
        mesh=mesh,
        scratch_types=[
            pltpu.VMEM((4, 128), jnp.int32),
            pltpu.VMEM((4, 128), jnp.float32),
            pltpu.VMEM((4, 128), jnp.int32),
            pltpu.VMEM((4, 128), jnp.int32),
            pltpu.VMEM((NW, 128), jnp.int32),
            pltpu.VMEM((128, D // 2), jnp.int32),
            pltpu.VMEM((128, D // 2), jnp.int32),
            pltpu.VMEM((NBP,), jnp.int32),
            pltpu.SemaphoreType.DMA,
            pltpu.SemaphoreType.DMA,
            pltpu.SemaphoreType.DMA,
            pltpu.SemaphoreType.DMA,
        ],
        compiler_params=pltpu.CompilerParams(needs_layout_passes=False),
    )
    return f(eid2, wgt2, cnt2, xp)


def _expert_body(bexp_ref, xs_ref, we_ref, be_ref, wsl_ref, ys_ref):
    xb = _unpack_bf16(xs_ref[...])
    h = jnp.dot(xb, we_ref[0], preferred_element_type=jnp.float32) + be_ref[0]
    h = 0.5 * h * (1.0 + lax.erf(h * 0.7071067811865476))
    ys_ref[...] = _pack_bf16(wsl_ref[0, 0][:, None] * h)


def _experts(bexp, xs, We_bf, be3, wsl3):
    grid_spec = pltpu.PrefetchScalarGridSpec(
        num_scalar_prefetch=1,
        grid=(NB,),
        in_specs=[
            pl.BlockSpec((M, D // 2), lambda b, bm: (b, 0)),
            pl.BlockSpec((1, D, D), lambda b, bm: (bm[b], 0, 0)),
            pl.BlockSpec((1, 1, D), lambda b, bm: (bm[b], 0, 0)),
            pl.BlockSpec((1, 1, M), lambda b, bm: (b, 0, 0)),
        ],
        out_specs=pl.BlockSpec((M, D // 2), lambda b, bm: (b, 0)),
    )
    return pl.pallas_call(
        _expert_body,
        grid_spec=grid_spec,
        out_shape=jax.ShapeDtypeStruct((NSLOT, D // 2), jnp.int32),
        compiler_params=pltpu.CompilerParams(
            dimension_semantics=("arbitrary",),
        ),
    )(bexp, xs, We_bf, be3, wsl3)


def _combine_body(ys_hbm, pos_hbm, ya_hbm, yb_hbm,
                  idx0_v, idx1_v, rows0_v, rows1_v, sg0, sg1, sw0, sw1):
    cid = lax.axis_index("c")
    sid = lax.axis_index("s")
    wid = sid * NC + cid

    jobs = []
    for h in range(2):
        dst = pl.ds(wid * TPW + h * 128, 128)
        jobs.append((wid * 2 + h, ya_hbm, dst))
        jobs.append((64 + wid * 2 + h, yb_hbm, dst))

    pltpu.sync_copy(pos_hbm.at[jobs[0][0]], idx0_v)
    g0 = pltpu.async_copy(ys_hbm.at[idx0_v], rows0_v, sg0)
    pltpu.sync_copy(pos_hbm.at[jobs[1][0]], idx1_v)
    g1 = pltpu.async_copy(ys_hbm.at[idx1_v], rows1_v, sg1)
    g0.wait()
    w0 = pltpu.async_copy(rows0_v, jobs[0][1].at[jobs[0][2]], sw0)
    g1.wait()
    w1 = pltpu.async_copy(rows1_v, jobs[1][1].at[jobs[1][2]], sw1)
    w0.wait()
    pltpu.sync_copy(pos_hbm.at[jobs[2][0]], idx0_v)
    g2 = pltpu.async_copy(ys_hbm.at[idx0_v], rows0_v, sg0)
    w1.wait()
    pltpu.sync_copy(pos_hbm.at[jobs[3][0]], idx1_v)
    g3 = pltpu.async_copy(ys_hbm.at[idx1_v], rows1_v, sg1)
    g2.wait()
    w2 = pltpu.async_copy(rows0_v, jobs[2][1].at[jobs[2][2]], sw0)
    g3.wait()
    w3 = pltpu.async_copy(rows1_v, jobs[3][1].at[jobs[3][2]], sw1)
    w2.wait()
    w3.wait()


def _combine(ys, pos2):
    mesh = plsc.VectorSubcoreMesh(core_axis_name="c", subcore_axis_name="s", num_cores=NC, num_subcores=NS)
    f = pl.kernel(
        _combine_body,
        out_type=[
            jax.ShapeDtypeStruct((T, D // 2), jnp.int32),
            jax.ShapeDtypeStruct((T, D // 2), jnp.int32),
        ],
        mesh=mesh,
        scratch_types=[
            pltpu.VMEM((128,), jnp.int32),
            pltpu.VMEM((128,), jnp.int32),
            pltpu.VMEM((128, D // 2), jnp.int32),
            pltpu.VMEM((128, D // 2), jnp.int32),
            pltpu.SemaphoreType.DMA,
            pltpu.SemaphoreType.DMA,
            pltpu.SemaphoreType.DMA,
            pltpu.SemaphoreType.DMA,
        ],
        compiler_params=pltpu.CompilerParams(needs_layout_passes=False),
    )
    return f(ys, pos2)


def _sum_body(a_ref, b_ref, o_ref):
    a = _unpack_bf16(a_ref[...]).astype(jnp.float32)
    b = _unpack_bf16(b_ref[...]).astype(jnp.float32)
    o_ref[...] = a + b


def _sum(ya, yb):
    return pl.pallas_call(
        _sum_body,
        grid=(T // RBLK,),
        in_specs=[
            pl.BlockSpec((RBLK, D // 2), lambda b: (b, 0)),
            pl.BlockSpec((RBLK, D // 2), lambda b: (b, 0)),
        ],
        out_specs=pl.BlockSpec((RBLK, D), lambda b: (b, 0)),
        out_shape=jax.ShapeDtypeStruct((T, D), jnp.float32),
        compiler_params=pltpu.CompilerParams(
            dimension_semantics=("parallel",),
        ),
    )(ya, yb)


@jax.jit
def kernel(x, Wr, We, be):
    Bx, Sx, Dx = x.shape
    xf = x.reshape(T, D)

    eid, wgt, cnt, xp = _router(xf, Wr)
    eid2 = eid.reshape(A // 128, 128)
    wgt2 = wgt.reshape(A // 128, 128)
    cnt2 = cnt.reshape(NW, 128)

    pos2, xs, wsl, bexp = _dispatch(eid2, wgt2, cnt2, xp)

    We_bf = We.astype(jnp.bfloat16)
    be3 = be.reshape(E, 1, D)
    wsl3 = wsl.reshape(NB, 1, M)
    ys = _experts(bexp, xs, We_bf, be3, wsl3)

    ya, yb = _combine(ys, pos2)
    out = _sum(ya, yb)
    return out.reshape(Bx, Sx, Dx)


# speedup vs baseline: 1.1785x; 1.0078x over previous
import functools

import jax
import jax.numpy as jnp
from jax import lax
from jax.experimental import pallas as pl
from jax.experimental.pallas import tpu as pltpu
from jax.experimental.pallas import tpu_sc as plsc

E = 8
D = 768
T = 8192
A = 2 * T
M = 256
NSLOT = A + E * M
NB = NSLOT // M
NBP = 80

NC = 2
NS = 16
NW = NC * NS
APW = A // NW
TPW = T // NW

RBLK = 512
_NEG_INF = -1e30


def _pack_bf16(h):
    u = lax.bitcast_convert_type(h, jnp.uint32)
    r = (u + 0x7FFF + ((u >> 16) & 1)) >> 16
    lo = r[:, : D // 2] & 0xFFFF
    hi = r[:, D // 2 :] & 0xFFFF
    return lax.bitcast_convert_type(lo | (hi << 16), jnp.int32)


def _unpack_bf16(p):
    v = lax.bitcast_convert_type(p, jnp.uint32)
    lo = (v & 0xFFFF).astype(jnp.uint16)
    hi = (v >> 16).astype(jnp.uint16)
    return jnp.concatenate(
        [
            lax.bitcast_convert_type(lo, jnp.bfloat16),
            lax.bitcast_convert_type(hi, jnp.bfloat16),
        ],
        axis=1,
    )


def _router_body(x_ref, wr_ref, eid_ref, wgt_ref, cnt_ref, xp_ref):
    xb = x_ref[...]
    xp_ref[...] = _pack_bf16(xb)
    logits = jnp.dot(xb, wr_ref[...])
    m1 = jnp.max(logits, axis=-1)
    i1 = jnp.argmax(logits, axis=-1)
    cols = lax.broadcasted_iota(jnp.int32, logits.shape, 1)
    masked = jnp.where(cols == i1[:, None], _NEG_INF, logits)
    m2 = jnp.max(masked, axis=-1)
    i2 = jnp.argmax(masked, axis=-1)
    w1 = 1.0 / (1.0 + jnp.exp(m2 - m1))
    eid_ref[0:1, :] = i1[None, :]
    eid_ref[1:2, :] = i2[None, :]
    wgt_ref[0:1, :] = w1[None, :]
    wgt_ref[1:2, :] = (1.0 - w1)[None, :]
    lanes = lax.broadcasted_iota(jnp.int32, (RBLK, 128), 1)
    cnt_ref[0, 0] = jnp.sum((i1[:, None] == lanes).astype(jnp.int32), axis=0)[None, :]
    cnt_ref[1, 0] = jnp.sum((i2[:, None] == lanes).astype(jnp.int32), axis=0)[None, :]


def _router(xf, Wr):
    nb = T // RBLK
    return pl.pallas_call(
        _router_body,
        grid=(nb,),
        in_specs=[
            pl.BlockSpec((RBLK, D), lambda b: (b, 0)),
            pl.BlockSpec((D, E), lambda b: (0, 0)),
        ],
        out_specs=[
            pl.BlockSpec((2, RBLK), lambda b: (0, b)),
            pl.BlockSpec((2, RBLK), lambda b: (0, b)),
            pl.BlockSpec((2, 1, 1, 128), lambda b: (0, b, 0, 0)),
            pl.BlockSpec((RBLK, D // 2), lambda b: (b, 0)),
        ],
        out_shape=[
            jax.ShapeDtypeStruct((2, T), jnp.int32),
            jax.ShapeDtypeStruct((2, T), jnp.float32),
            jax.ShapeDtypeStruct((2, nb, 1, 128), jnp.int32),
            jax.ShapeDtypeStruct((T, D // 2), jnp.int32),
        ],
        compiler_params=pltpu.CompilerParams(
            dimension_semantics=("parallel",),
        ),
    )(xf, Wr)


def _iota16():
    return lax.iota(jnp.int32, 16)


def _lane(v, e):
    return lax.squeeze(lax.slice(v, (e,), (e + 1,)), (0,))


def _dispatch_body(eid_hbm, wgt_hbm, cnt_hbm, xp_hbm,
                  pos_hbm, xs_hbm, wsl_hbm, bexp_hbm,
                  eid_v, wgt_v, slots_v, tok_v, cnt_v, rows0_v, rows1_v, bexp_v,
                  sg0, sg1, ss0, ss1, sw):
    cid = lax.axis_index("c")
    sid = lax.axis_index("s")
    wid = sid * NC + cid

    pltpu.sync_copy(eid_hbm.at[pl.ds(wid * 4, 4)], eid_v)
    pltpu.sync_copy(wgt_hbm.at[pl.ds(wid * 4, 4)], wgt_v)
    pltpu.sync_copy(cnt_hbm, cnt_v)

    for j in range(APW // 16):
        r, o = j // 8, (j % 8) * 16
        flat = wid * APW + j * 16
        tok_v[r, pl.ds(o, 16)] = (flat + _iota16()) & (T - 1)
    g0 = pltpu.async_copy(xp_hbm.at[tok_v.at[0]], rows0_v, sg0)
    g1 = pltpu.async_copy(xp_hbm.at[tok_v.at[1]], rows1_v, sg1)

    zero = jnp.zeros((16,), jnp.int32)
    tot = zero
    pre = zero
    wid_v = jnp.full((16,), wid, jnp.int32)
    for r in range(NW):
        row = cnt_v[r, pl.ds(0, 16)]
        tot = tot + row
        r_v = jnp.full((16,), r, jnp.int32)
        pre = pre + jnp.where(r_v < wid_v, row, zero)
    aligned = ((tot + (M - 1)) >> 8) << 8
    base = plsc.cumsum(aligned) - aligned
    start = base + pre

    starts = [_lane(start, e) for e in range(E)]
    for j in range(APW // 16):
        r, o = j // 8, (j % 8) * 16
        v = eid_v[r, pl.ds(o, 16)]
        ms = [v == e for e in range(E)]
        cs = [plsc.cumsum(m.astype(jnp.int32)) for m in ms]
        pcs = [plsc.all_reduce_population_count(m) for m in ms]
        slot = zero
        for e in range(E):
            slot = jnp.where(ms[e], starts[e] + cs[e] - 1, slot)
            starts[e] = starts[e] + pcs[e]
        slots_v[r, pl.ds(o, 16)] = slot

    pltpu.sync_copy(slots_v, pos_hbm.at[pl.ds(wid * 4, 4)])

    w0 = pltpu.async_copy(wgt_v.at[0], wsl_hbm.at[slots_v.at[0]], sw)
    w1 = pltpu.async_copy(wgt_v.at[1], wsl_hbm.at[slots_v.at[1]], sw)
    w2 = pltpu.async_copy(wgt_v.at[2], wsl_hbm.at[slots_v.at[2]], sw)
    w3 = pltpu.async_copy(wgt_v.at[3], wsl_hbm.at[slots_v.at[3]], sw)
    g0.wait()
    s0 = pltpu.async_copy(rows0_v, xs_hbm.at[slots_v.at[0]], ss0)
    g1.wait()
    s1 = pltpu.async_copy(rows1_v, xs_hbm.at[slots_v.at[1]], ss1)
    s0.wait()
    g2 = pltpu.async_copy(xp_hbm.at[tok_v.at[2]], rows0_v, sg0)
    s1.wait()
    g3 = pltpu.async_copy(xp_hbm.at[tok_v.at[3]], rows1_v, sg1)
    g2.wait()
    s2 = pltpu.async_copy(rows0_v, xs_hbm.at[slots_v.at[2]], ss0)
    g3.wait()
    s3 = pltpu.async_copy(rows1_v, xs_hbm.at[slots_v.at[3]], ss1)
    w0.wait()
    w1.wait()
    w2.wait()
    w3.wait()
    s2.wait()
    s3.wait()

    @pl.when(wid == 0)
    def _():
        for q in range(NBP // 16):
            bstart = (_iota16() + q * 16) * M
            acc = jnp.zeros((16,), jnp.int32)
            for e in range(1, E):
                acc = acc + (bstart >= _lane(base, e)).astype(jnp.int32)
            bexp_v[pl.ds(q * 16, 16)] = acc
        pltpu.sync_copy(bexp_v, bexp_hbm)


def _dispatch(eid2, wgt2, cnt2, xp):
    mesh = plsc.VectorSubcoreMesh(core_axis_name="c", subcore_axis_name="s", num_cores=NC, num_subcores=NS)
    f = pl.kernel(
        _dispatch_body,
        out_type=[
            jax.ShapeDtypeStruct((A // 128, 128), jnp.int32),
            jax.ShapeDtypeStruct((NSLOT, D // 2), jnp.int32),
            jax.ShapeDtypeStruct((NSLOT,), jnp.float32),
            jax.ShapeDtypeStruct((NBP,), jnp.int32),
        ],
        mesh=mesh,
        scratch_types=[
            pltpu.VMEM((4, 128), jnp.int32),
            pltpu.VMEM((4, 128), jnp.float32),
            pltpu.VMEM((4, 128), jnp.int32),
            pltpu.VMEM((4, 128), jnp.int32),
            pltpu.VMEM((NW, 128), jnp.int32),
            pltpu.VMEM((128, D // 2), jnp.int32),
            pltpu.VMEM((128, D // 2), jnp.int32),
            pltpu.VMEM((NBP,), jnp.int32),
            pltpu.SemaphoreType.DMA,
            pltpu.SemaphoreType.DMA,
            pltpu.SemaphoreType.DMA,
            pltpu.SemaphoreType.DMA,
            pltpu.SemaphoreType.DMA,
        ],
        compiler_params=pltpu.CompilerParams(needs_layout_passes=False),
    )
    return f(eid2, wgt2, cnt2, xp)


def _expert_body(bexp_ref, xs_ref, we_ref, be_ref, wsl_ref, ys_ref):
    xb = _unpack_bf16(xs_ref[...])
    h = jnp.dot(xb, we_ref[0], preferred_element_type=jnp.float32) + be_ref[0]
    h = 0.5 * h * (1.0 + lax.erf(h * 0.7071067811865476))
    ys_ref[...] = _pack_bf16(wsl_ref[0, 0][:, None] * h)


def _experts(bexp, xs, We_bf, be3, wsl3):
    grid_spec = pltpu.PrefetchScalarGridSpec(
        num_scalar_prefetch=1,
        grid=(NB,),
        in_specs=[
            pl.BlockSpec((M, D // 2), lambda b, bm: (b, 0)),
            pl.BlockSpec((1, D, D), lambda b, bm: (bm[b], 0, 0)),
            pl.BlockSpec((1, 1, D), lambda b, bm: (bm[b], 0, 0)),
            pl.BlockSpec((1, 1, M), lambda b, bm: (b, 0, 0)),
        ],
        out_specs=pl.BlockSpec((M, D // 2), lambda b, bm: (b, 0)),
    )
    return pl.pallas_call(
        _expert_body,
        grid_spec=grid_spec,
        out_shape=jax.ShapeDtypeStruct((NSLOT, D // 2), jnp.int32),
        compiler_params=pltpu.CompilerParams(
            dimension_semantics=("arbitrary",),
        ),
    )(bexp, xs, We_bf, be3, wsl3)


def _combine_body(ys_hbm, pos_hbm, ya_hbm, yb_hbm,
                  idx0_v, idx1_v, rows0_v, rows1_v, sg0, sg1, sw0, sw1):
    cid = lax.axis_index("c")
    sid = lax.axis_index("s")
    wid = sid * NC + cid

    jobs = []
    for h in range(2):
        dst = pl.ds(wid * TPW + h * 128, 128)
        jobs.append((wid * 2 + h, ya_hbm, dst))
        jobs.append((64 + wid * 2 + h, yb_hbm, dst))

    pltpu.sync_copy(pos_hbm.at[jobs[0][0]], idx0_v)
    g0 = pltpu.async_copy(ys_hbm.at[idx0_v], rows0_v, sg0)
    pltpu.sync_copy(pos_hbm.at[jobs[1][0]], idx1_v)
    g1 = pltpu.async_copy(ys_hbm.at[idx1_v], rows1_v, sg1)
    g0.wait()
    w0 = pltpu.async_copy(rows0_v, jobs[0][1].at[jobs[0][2]], sw0)
    g1.wait()
    w1 = pltpu.async_copy(rows1_v, jobs[1][1].at[jobs[1][2]], sw1)
    w0.wait()
    pltpu.sync_copy(pos_hbm.at[jobs[2][0]], idx0_v)
    g2 = pltpu.async_copy(ys_hbm.at[idx0_v], rows0_v, sg0)
    w1.wait()
    pltpu.sync_copy(pos_hbm.at[jobs[3][0]], idx1_v)
    g3 = pltpu.async_copy(ys_hbm.at[idx1_v], rows1_v, sg1)
    g2.wait()
    w2 = pltpu.async_copy(rows0_v, jobs[2][1].at[jobs[2][2]], sw0)
    g3.wait()
    w3 = pltpu.async_copy(rows1_v, jobs[3][1].at[jobs[3][2]], sw1)
    w2.wait()
    w3.wait()


def _combine(ys, pos2):
    mesh = plsc.VectorSubcoreMesh(core_axis_name="c", subcore_axis_name="s", num_cores=NC, num_subcores=NS)
    f = pl.kernel(
        _combine_body,
        out_type=[
            jax.ShapeDtypeStruct((T, D // 2), jnp.int32),
            jax.ShapeDtypeStruct((T, D // 2), jnp.int32),
        ],
        mesh=mesh,
        scratch_types=[
            pltpu.VMEM((128,), jnp.int32),
            pltpu.VMEM((128,), jnp.int32),
            pltpu.VMEM((128, D // 2), jnp.int32),
            pltpu.VMEM((128, D // 2), jnp.int32),
            pltpu.SemaphoreType.DMA,
            pltpu.SemaphoreType.DMA,
            pltpu.SemaphoreType.DMA,
            pltpu.SemaphoreType.DMA,
        ],
        compiler_params=pltpu.CompilerParams(needs_layout_passes=False),
    )
    return f(ys, pos2)


def _sum_body(a_ref, b_ref, o_ref):
    a = _unpack_bf16(a_ref[...]).astype(jnp.float32)
    b = _unpack_bf16(b_ref[...]).astype(jnp.float32)
    o_ref[...] = a + b


def _sum(ya, yb):
    return pl.pallas_call(
        _sum_body,
        grid=(T // RBLK,),
        in_specs=[
            pl.BlockSpec((RBLK, D // 2), lambda b: (b, 0)),
            pl.BlockSpec((RBLK, D // 2), lambda b: (b, 0)),
        ],
        out_specs=pl.BlockSpec((RBLK, D), lambda b: (b, 0)),
        out_shape=jax.ShapeDtypeStruct((T, D), jnp.float32),
        compiler_params=pltpu.CompilerParams(
            dimension_semantics=("parallel",),
        ),
    )(ya, yb)


@jax.jit
def kernel(x, Wr, We, be):
    Bx, Sx, Dx = x.shape
    xf = x.reshape(T, D)

    eid, wgt, cnt, xp = _router(xf, Wr)
    eid2 = eid.reshape(A // 128, 128)
    wgt2 = wgt.reshape(A // 128, 128)
    cnt2 = cnt.reshape(NW, 128)

    pos2, xs, wsl, bexp = _dispatch(eid2, wgt2, cnt2, xp)

    We_bf = We.astype(jnp.bfloat16)
    be3 = be.reshape(E, 1, D)
    wsl3 = wsl.reshape(NB, 1, M)
    ys = _experts(bexp, xs, We_bf, be3, wsl3)

    ya, yb = _combine(ys, pos2)
    out = _sum(ya, yb)
    return out.reshape(Bx, Sx, Dx)


# speedup vs baseline: 1.4648x; 1.2430x over previous
import functools

import jax
import jax.numpy as jnp
from jax import lax
from jax.experimental import pallas as pl
from jax.experimental.pallas import tpu as pltpu
from jax.experimental.pallas import tpu_sc as plsc

E = 8
D = 768
T = 8192
A = 2 * T
M = 256
NSLOT = A + E * M
NB = NSLOT // M
NBP = 80

NC = 2
NS = 16
NW = NC * NS
APW = A // NW
TPW = T // NW

RBLK = 512
_NEG_INF = -1e30


def _pack_bf16(h):
    u = lax.bitcast_convert_type(h, jnp.uint32)
    r = (u + 0x7FFF + ((u >> 16) & 1)) >> 16
    lo = r[:, : D // 2] & 0xFFFF
    hi = r[:, D // 2 :] & 0xFFFF
    return lax.bitcast_convert_type(lo | (hi << 16), jnp.int32)


def _unpack_bf16(p):
    v = lax.bitcast_convert_type(p, jnp.uint32)
    lo = (v & 0xFFFF).astype(jnp.uint16)
    hi = (v >> 16).astype(jnp.uint16)
    return jnp.concatenate(
        [
            lax.bitcast_convert_type(lo, jnp.bfloat16),
            lax.bitcast_convert_type(hi, jnp.bfloat16),
        ],
        axis=1,
    )


def _router_body(x_ref, wr_ref, eid_ref, wgt_ref, cnt_ref, xp_ref):
    xb = x_ref[...]
    xp_ref[...] = _pack_bf16(xb)
    logits = jnp.dot(xb, wr_ref[...])
    m1 = jnp.max(logits, axis=-1)
    i1 = jnp.argmax(logits, axis=-1)
    cols = lax.broadcasted_iota(jnp.int32, logits.shape, 1)
    masked = jnp.where(cols == i1[:, None], _NEG_INF, logits)
    m2 = jnp.max(masked, axis=-1)
    i2 = jnp.argmax(masked, axis=-1)
    w1 = 1.0 / (1.0 + jnp.exp(m2 - m1))
    eid_ref[0:1, :] = i1[None, :]
    eid_ref[1:2, :] = i2[None, :]
    wgt_ref[0:1, :] = w1[None, :]
    wgt_ref[1:2, :] = (1.0 - w1)[None, :]
    lanes = lax.broadcasted_iota(jnp.int32, (RBLK, 128), 1)
    cnt_ref[0, 0] = jnp.sum((i1[:, None] == lanes).astype(jnp.int32), axis=0)[None, :]
    cnt_ref[1, 0] = jnp.sum((i2[:, None] == lanes).astype(jnp.int32), axis=0)[None, :]


def _router(xf, Wr):
    nb = T // RBLK
    return pl.pallas_call(
        _router_body,
        grid=(nb,),
        in_specs=[
            pl.BlockSpec((RBLK, D), lambda b: (b, 0)),
            pl.BlockSpec((D, E), lambda b: (0, 0)),
        ],
        out_specs=[
            pl.BlockSpec((2, RBLK), lambda b: (0, b)),
            pl.BlockSpec((2, RBLK), lambda b: (0, b)),
            pl.BlockSpec((2, 1, 1, 128), lambda b: (0, b, 0, 0)),
            pl.BlockSpec((RBLK, D // 2), lambda b: (b, 0)),
        ],
        out_shape=[
            jax.ShapeDtypeStruct((2, T), jnp.int32),
            jax.ShapeDtypeStruct((2, T), jnp.float32),
            jax.ShapeDtypeStruct((2, nb, 1, 128), jnp.int32),
            jax.ShapeDtypeStruct((T, D // 2), jnp.int32),
        ],
        compiler_params=pltpu.CompilerParams(
            dimension_semantics=("parallel",),
        ),
    )(xf, Wr)


def _iota16():
    return lax.iota(jnp.int32, 16)


def _lane(v, e):
    return lax.squeeze(lax.slice(v, (e,), (e + 1,)), (0,))


def _dispatch_body(eid_hbm, cnt_hbm, xp_hbm,
                  pos_hbm, xs_hbm, bexp_hbm,
                  eid_v, slots_v, tok_v, cnt_v, rows0_v, rows1_v, bexp_v,
                  sg0, sg1, ss0, ss1):
    cid = lax.axis_index("c")
    sid = lax.axis_index("s")
    wid = sid * NC + cid

    pltpu.sync_copy(eid_hbm.at[pl.ds(wid * 4, 4)], eid_v)
    pltpu.sync_copy(cnt_hbm, cnt_v)

    for j in range(APW // 16):
        r, o = j // 8, (j % 8) * 16
        flat = wid * APW + j * 16
        tok_v[r, pl.ds(o, 16)] = (flat + _iota16()) & (T - 1)
    g0 = pltpu.async_copy(xp_hbm.at[tok_v.at[0]], rows0_v, sg0)
    g1 = pltpu.async_copy(xp_hbm.at[tok_v.at[1]], rows1_v, sg1)

    zero = jnp.zeros((16,), jnp.int32)
    tot = zero
    pre = zero
    wid_v = jnp.full((16,), wid, jnp.int32)
    for r in range(NW):
        row = cnt_v[r, pl.ds(0, 16)]
        tot = tot + row
        r_v = jnp.full((16,), r, jnp.int32)
        pre = pre + jnp.where(r_v < wid_v, row, zero)
    aligned = ((tot + (M - 1)) >> 8) << 8
    base = plsc.cumsum(aligned) - aligned
    start = base + pre

    starts = [_lane(start, e) for e in range(E)]
    for j in range(APW // 16):
        r, o = j // 8, (j % 8) * 16
        v = eid_v[r, pl.ds(o, 16)]
        ms = [v == e for e in range(E)]
        cs = [plsc.cumsum(m.astype(jnp.int32)) for m in ms]
        pcs = [plsc.all_reduce_population_count(m) for m in ms]
        slot = zero
        for e in range(E):
            slot = jnp.where(ms[e], starts[e] + cs[e] - 1, slot)
            starts[e] = starts[e] + pcs[e]
        slots_v[r, pl.ds(o, 16)] = slot

    pltpu.sync_copy(slots_v, pos_hbm.at[pl.ds(wid * 4, 4)])

    g0.wait()
    s0 = pltpu.async_copy(rows0_v, xs_hbm.at[slots_v.at[0]], ss0)
    g1.wait()
    s1 = pltpu.async_copy(rows1_v, xs_hbm.at[slots_v.at[1]], ss1)
    s0.wait()
    g2 = pltpu.async_copy(xp_hbm.at[tok_v.at[2]], rows0_v, sg0)
    s1.wait()
    g3 = pltpu.async_copy(xp_hbm.at[tok_v.at[3]], rows1_v, sg1)
    g2.wait()
    s2 = pltpu.async_copy(rows0_v, xs_hbm.at[slots_v.at[2]], ss0)
    g3.wait()
    s3 = pltpu.async_copy(rows1_v, xs_hbm.at[slots_v.at[3]], ss1)
    s2.wait()
    s3.wait()

    @pl.when(wid == 0)
    def _():
        for q in range(NBP // 16):
            bstart = (_iota16() + q * 16) * M
            acc = jnp.zeros((16,), jnp.int32)
            for e in range(1, E):
                acc = acc + (bstart >= _lane(base, e)).astype(jnp.int32)
            bexp_v[pl.ds(q * 16, 16)] = acc
        pltpu.sync_copy(bexp_v, bexp_hbm)


def _dispatch(eid2, cnt2, xp):
    mesh = plsc.VectorSubcoreMesh(core_axis_name="c", subcore_axis_name="s", num_cores=NC, num_subcores=NS)
    f = pl.kernel(
        _dispatch_body,
        out_type=[
            jax.ShapeDtypeStruct((A // 128, 128), jnp.int32),
            jax.ShapeDtypeStruct((NSLOT, D // 2), jnp.int32),
            jax.ShapeDtypeStruct((NBP,), jnp.int32),
        ],
        mesh=mesh,
        scratch_types=[
            pltpu.VMEM((4, 128), jnp.int32),
            pltpu.VMEM((4, 128), jnp.int32),
            pltpu.VMEM((4, 128), jnp.int32),
            pltpu.VMEM((NW, 128), jnp.int32),
            pltpu.VMEM((128, D // 2), jnp.int32),
            pltpu.VMEM((128, D // 2), jnp.int32),
            pltpu.VMEM((NBP,), jnp.int32),
            pltpu.SemaphoreType.DMA,
            pltpu.SemaphoreType.DMA,
            pltpu.SemaphoreType.DMA,
            pltpu.SemaphoreType.DMA,
        ],
        compiler_params=pltpu.CompilerParams(needs_layout_passes=False),
    )
    return f(eid2, cnt2, xp)


def _expert_body(bexp_ref, xs_ref, we_ref, be_ref, ys_ref):
    xb = _unpack_bf16(xs_ref[...])
    h = jnp.dot(xb, we_ref[0], preferred_element_type=jnp.float32) + be_ref[0]
    h = 0.5 * h * (1.0 + lax.erf(h * 0.7071067811865476))
    ys_ref[...] = _pack_bf16(h)


def _experts(bexp, xs, We_bf, be3):
    grid_spec = pltpu.PrefetchScalarGridSpec(
        num_scalar_prefetch=1,
        grid=(NB,),
        in_specs=[
            pl.BlockSpec((M, D // 2), lambda b, bm: (b, 0)),
            pl.BlockSpec((1, D, D), lambda b, bm: (bm[b], 0, 0)),
            pl.BlockSpec((1, 1, D), lambda b, bm: (bm[b], 0, 0)),
        ],
        out_specs=pl.BlockSpec((M, D // 2), lambda b, bm: (b, 0)),
    )
    return pl.pallas_call(
        _expert_body,
        grid_spec=grid_spec,
        out_shape=jax.ShapeDtypeStruct((NSLOT, D // 2), jnp.int32),
        compiler_params=pltpu.CompilerParams(
            dimension_semantics=("arbitrary",),
        ),
    )(bexp, xs, We_bf, be3)


def _combine_body(ys_hbm, pos_hbm, ya_hbm, yb_hbm,
                  idx0_v, idx1_v, rows0_v, rows1_v, sg0, sg1, sw0, sw1):
    cid = lax.axis_index("c")
    sid = lax.axis_index("s")
    wid = sid * NC + cid

    jobs = []
    for h in range(2):
        dst = pl.ds(wid * TPW + h * 128, 128)
        jobs.append((wid * 2 + h, ya_hbm, dst))
        jobs.append((64 + wid * 2 + h, yb_hbm, dst))

    pltpu.sync_copy(pos_hbm.at[jobs[0][0]], idx0_v)
    g0 = pltpu.async_copy(ys_hbm.at[idx0_v], rows0_v, sg0)
    pltpu.sync_copy(pos_hbm.at[jobs[1][0]], idx1_v)
    g1 = pltpu.async_copy(ys_hbm.at[idx1_v], rows1_v, sg1)
    g0.wait()
    w0 = pltpu.async_copy(rows0_v, jobs[0][1].at[jobs[0][2]], sw0)
    g1.wait()
    w1 = pltpu.async_copy(rows1_v, jobs[1][1].at[jobs[1][2]], sw1)
    w0.wait()
    pltpu.sync_copy(pos_hbm.at[jobs[2][0]], idx0_v)
    g2 = pltpu.async_copy(ys_hbm.at[idx0_v], rows0_v, sg0)
    w1.wait()
    pltpu.sync_copy(pos_hbm.at[jobs[3][0]], idx1_v)
    g3 = pltpu.async_copy(ys_hbm.at[idx1_v], rows1_v, sg1)
    g2.wait()
    w2 = pltpu.async_copy(rows0_v, jobs[2][1].at[jobs[2][2]], sw0)
    g3.wait()
    w3 = pltpu.async_copy(rows1_v, jobs[3][1].at[jobs[3][2]], sw1)
    w2.wait()
    w3.wait()


def _combine(ys, pos2):
    mesh = plsc.VectorSubcoreMesh(core_axis_name="c", subcore_axis_name="s", num_cores=NC, num_subcores=NS)
    f = pl.kernel(
        _combine_body,
        out_type=[
            jax.ShapeDtypeStruct((T, D // 2), jnp.int32),
            jax.ShapeDtypeStruct((T, D // 2), jnp.int32),
        ],
        mesh=mesh,
        scratch_types=[
            pltpu.VMEM((128,), jnp.int32),
            pltpu.VMEM((128,), jnp.int32),
            pltpu.VMEM((128, D // 2), jnp.int32),
            pltpu.VMEM((128, D // 2), jnp.int32),
            pltpu.SemaphoreType.DMA,
            pltpu.SemaphoreType.DMA,
            pltpu.SemaphoreType.DMA,
            pltpu.SemaphoreType.DMA,
        ],
        compiler_params=pltpu.CompilerParams(needs_layout_passes=False),
    )
    return f(ys, pos2)


def _sum_body(a_ref, b_ref, w_ref, o_ref):
    a = _unpack_bf16(a_ref[...]).astype(jnp.float32)
    b = _unpack_bf16(b_ref[...]).astype(jnp.float32)
    o_ref[...] = w_ref[0, :][:, None] * a + w_ref[1, :][:, None] * b


def _sum(ya, yb, wgt):
    return pl.pallas_call(
        _sum_body,
        grid=(T // RBLK,),
        in_specs=[
            pl.BlockSpec((RBLK, D // 2), lambda b: (b, 0)),
            pl.BlockSpec((RBLK, D // 2), lambda b: (b, 0)),
            pl.BlockSpec((2, RBLK), lambda b: (0, b)),
        ],
        out_specs=pl.BlockSpec((RBLK, D), lambda b: (b, 0)),
        out_shape=jax.ShapeDtypeStruct((T, D), jnp.float32),
        compiler_params=pltpu.CompilerParams(
            dimension_semantics=("parallel",),
        ),
    )(ya, yb, wgt)


@jax.jit
def kernel(x, Wr, We, be):
    Bx, Sx, Dx = x.shape
    xf = x.reshape(T, D)

    eid, wgt, cnt, xp = _router(xf, Wr)
    eid2 = eid.reshape(A // 128, 128)
    cnt2 = cnt.reshape(NW, 128)

    pos2, xs, bexp = _dispatch(eid2, cnt2, xp)

    We_bf = We.astype(jnp.bfloat16)
    be3 = be.reshape(E, 1, D)
    ys = _experts(bexp, xs, We_bf, be3)

    ya, yb = _combine(ys, pos2)
    out = _sum(ya, yb, wgt)
    return out.reshape(Bx, Sx, Dx)


# speedup vs baseline: 1.5786x; 1.0777x over previous
import functools

import jax
import jax.numpy as jnp
from jax import lax
from jax.experimental import pallas as pl
from jax.experimental.pallas import tpu as pltpu
from jax.experimental.pallas import tpu_sc as plsc

E = 8
D = 768
T = 8192
A = 2 * T
M = 512
_MSHIFT = 9
NSLOT = A + E * M
NB = NSLOT // M
NBP = 48

NC = 2
NS = 16
NW = NC * NS
APW = A // NW
TPW = T // NW

RBLK = 512
_NEG_INF = -1e30


def _pack_bf16(h):
    u = lax.bitcast_convert_type(h, jnp.uint32)
    r = (u + 0x7FFF + ((u >> 16) & 1)) >> 16
    lo = r[:, : D // 2] & 0xFFFF
    hi = r[:, D // 2 :] & 0xFFFF
    return lax.bitcast_convert_type(lo | (hi << 16), jnp.int32)


def _unpack_bf16(p):
    v = lax.bitcast_convert_type(p, jnp.uint32)
    lo = (v & 0xFFFF).astype(jnp.uint16)
    hi = (v >> 16).astype(jnp.uint16)
    return jnp.concatenate(
        [
            lax.bitcast_convert_type(lo, jnp.bfloat16),
            lax.bitcast_convert_type(hi, jnp.bfloat16),
        ],
        axis=1,
    )


def _router_body(x_ref, wr_ref, eid_ref, wgt_ref, cnt_ref, xp_ref):
    xb = x_ref[...]
    xp_ref[...] = _pack_bf16(xb)
    logits = jnp.dot(xb, wr_ref[...])
    m1 = jnp.max(logits, axis=-1)
    i1 = jnp.argmax(logits, axis=-1)
    cols = lax.broadcasted_iota(jnp.int32, logits.shape, 1)
    masked = jnp.where(cols == i1[:, None], _NEG_INF, logits)
    m2 = jnp.max(masked, axis=-1)
    i2 = jnp.argmax(masked, axis=-1)
    w1 = 1.0 / (1.0 + jnp.exp(m2 - m1))
    eid_ref[0:1, :] = i1[None, :]
    eid_ref[1:2, :] = i2[None, :]
    wgt_ref[0:1, :] = w1[None, :]
    wgt_ref[1:2, :] = (1.0 - w1)[None, :]
    lanes = lax.broadcasted_iota(jnp.int32, (RBLK, 128), 1)
    cnt_ref[0, 0] = jnp.sum((i1[:, None] == lanes).astype(jnp.int32), axis=0)[None, :]
    cnt_ref[1, 0] = jnp.sum((i2[:, None] == lanes).astype(jnp.int32), axis=0)[None, :]


def _router(xf, Wr):
    nb = T // RBLK
    return pl.pallas_call(
        _router_body,
        grid=(nb,),
        in_specs=[
            pl.BlockSpec((RBLK, D), lambda b: (b, 0)),
            pl.BlockSpec((D, E), lambda b: (0, 0)),
        ],
        out_specs=[
            pl.BlockSpec((2, RBLK), lambda b: (0, b)),
            pl.BlockSpec((2, RBLK), lambda b: (0, b)),
            pl.BlockSpec((2, 1, 1, 128), lambda b: (0, b, 0, 0)),
            pl.BlockSpec((RBLK, D // 2), lambda b: (b, 0)),
        ],
        out_shape=[
            jax.ShapeDtypeStruct((2, T), jnp.int32),
            jax.ShapeDtypeStruct((2, T), jnp.float32),
            jax.ShapeDtypeStruct((2, nb, 1, 128), jnp.int32),
            jax.ShapeDtypeStruct((T, D // 2), jnp.int32),
        ],
        compiler_params=pltpu.CompilerParams(
            dimension_semantics=("parallel",),
        ),
    )(xf, Wr)


def _iota16():
    return lax.iota(jnp.int32, 16)


def _lane(v, e):
    return lax.squeeze(lax.slice(v, (e,), (e + 1,)), (0,))


def _dispatch_body(eid_hbm, cnt_hbm, xp_hbm,
                  pos_hbm, xs_hbm, bexp_hbm,
                  eid_v, slots_v, tok_v, cnt_v, rows0_v, rows1_v, bexp_v,
                  sg0, sg1, ss0, ss1):
    cid = lax.axis_index("c")
    sid = lax.axis_index("s")
    wid = sid * NC + cid

    pltpu.sync_copy(eid_hbm.at[pl.ds(wid * 4, 4)], eid_v)
    pltpu.sync_copy(cnt_hbm, cnt_v)

    for j in range(APW // 16):
        r, o = j // 8, (j % 8) * 16
        flat = wid * APW + j * 16
        tok_v[r, pl.ds(o, 16)] = (flat + _iota16()) & (T - 1)
    g0 = pltpu.async_copy(xp_hbm.at[tok_v.at[0]], rows0_v, sg0)
    g1 = pltpu.async_copy(xp_hbm.at[tok_v.at[1]], rows1_v, sg1)

    zero = jnp.zeros((16,), jnp.int32)
    tot = zero
    pre = zero
    wid_v = jnp.full((16,), wid, jnp.int32)
    for r in range(NW):
        row = cnt_v[r, pl.ds(0, 16)]
        tot = tot + row
        r_v = jnp.full((16,), r, jnp.int32)
        pre = pre + jnp.where(r_v < wid_v, row, zero)
    aligned = ((tot + (M - 1)) >> _MSHIFT) << _MSHIFT
    base = plsc.cumsum(aligned) - aligned
    start = base + pre

    starts = [_lane(start, e) for e in range(E)]
    for j in range(APW // 16):
        r, o = j // 8, (j % 8) * 16
        v = eid_v[r, pl.ds(o, 16)]
        ms = [v == e for e in range(E)]
        cs = [plsc.cumsum(m.astype(jnp.int32)) for m in ms]
        pcs = [plsc.all_reduce_population_count(m) for m in ms]
        slot = zero
        for e in range(E):
            slot = jnp.where(ms[e], starts[e] + cs[e] - 1, slot)
            starts[e] = starts[e] + pcs[e]
        slots_v[r, pl.ds(o, 16)] = slot

    pltpu.sync_copy(slots_v, pos_hbm.at[pl.ds(wid * 4, 4)])

    g0.wait()
    s0 = pltpu.async_copy(rows0_v, xs_hbm.at[slots_v.at[0]], ss0)
    g1.wait()
    s1 = pltpu.async_copy(rows1_v, xs_hbm.at[slots_v.at[1]], ss1)
    s0.wait()
    g2 = pltpu.async_copy(xp_hbm.at[tok_v.at[2]], rows0_v, sg0)
    s1.wait()
    g3 = pltpu.async_copy(xp_hbm.at[tok_v.at[3]], rows1_v, sg1)
    g2.wait()
    s2 = pltpu.async_copy(rows0_v, xs_hbm.at[slots_v.at[2]], ss0)
    g3.wait()
    s3 = pltpu.async_copy(rows1_v, xs_hbm.at[slots_v.at[3]], ss1)
    s2.wait()
    s3.wait()

    @pl.when(wid == 0)
    def _():
        for q in range(NBP // 16):
            bstart = (_iota16() + q * 16) * M
            acc = jnp.zeros((16,), jnp.int32)
            for e in range(1, E):
                acc = acc + (bstart >= _lane(base, e)).astype(jnp.int32)
            bexp_v[pl.ds(q * 16, 16)] = acc
        pltpu.sync_copy(bexp_v, bexp_hbm)


def _dispatch(eid2, cnt2, xp):
    mesh = plsc.VectorSubcoreMesh(core_axis_name="c", subcore_axis_name="s", num_cores=NC, num_subcores=NS)
    f = pl.kernel(
        _dispatch_body,
        out_type=[
            jax.ShapeDtypeStruct((A // 128, 128), jnp.int32),
            jax.ShapeDtypeStruct((NSLOT, D // 2), jnp.int32),
            jax.ShapeDtypeStruct((NBP,), jnp.int32),
        ],
        mesh=mesh,
        scratch_types=[
            pltpu.VMEM((4, 128), jnp.int32),
            pltpu.VMEM((4, 128), jnp.int32),
            pltpu.VMEM((4, 128), jnp.int32),
            pltpu.VMEM((NW, 128), jnp.int32),
            pltpu.VMEM((128, D // 2), jnp.int32),
            pltpu.VMEM((128, D // 2), jnp.int32),
            pltpu.VMEM((NBP,), jnp.int32),
            pltpu.SemaphoreType.DMA,
            pltpu.SemaphoreType.DMA,
            pltpu.SemaphoreType.DMA,
            pltpu.SemaphoreType.DMA,
        ],
        compiler_params=pltpu.CompilerParams(needs_layout_passes=False),
    )
    return f(eid2, cnt2, xp)


def _expert_body(bexp_ref, xs_ref, we_ref, be_ref, ys_ref):
    xb = _unpack_bf16(xs_ref[...])
    h = jnp.dot(xb, we_ref[0], preferred_element_type=jnp.float32) + be_ref[0]
    h = 0.5 * h * (1.0 + lax.erf(h * 0.7071067811865476))
    ys_ref[...] = _pack_bf16(h)


def _experts(bexp, xs, We_bf, be3):
    grid_spec = pltpu.PrefetchScalarGridSpec(
        num_scalar_prefetch=1,
        grid=(NB,),
        in_specs=[
            pl.BlockSpec((M, D // 2), lambda b, bm: (b, 0)),
            pl.BlockSpec((1, D, D), lambda b, bm: (bm[b], 0, 0)),
            pl.BlockSpec((1, 1, D), lambda b, bm: (bm[b], 0, 0)),
        ],
        out_specs=pl.BlockSpec((M, D // 2), lambda b, bm: (b, 0)),
    )
    return pl.pallas_call(
        _expert_body,
        grid_spec=grid_spec,
        out_shape=jax.ShapeDtypeStruct((NSLOT, D // 2), jnp.int32),
        compiler_params=pltpu.CompilerParams(
            dimension_semantics=("arbitrary",),
        ),
    )(bexp, xs, We_bf, be3)


def _combine_body(ys_hbm, pos_hbm, ya_hbm, yb_hbm,
                  idx0_v, idx1_v, rows0_v, rows1_v, sg0, sg1, sw0, sw1):
    cid = lax.axis_index("c")
    sid = lax.axis_index("s")
    wid = sid * NC + cid

    jobs = []
    for h in range(2):
        dst = pl.ds(wid * TPW + h * 128, 128)
        jobs.append((wid * 2 + h, ya_hbm, dst))
        jobs.append((64 + wid * 2 + h, yb_hbm, dst))

    pltpu.sync_copy(pos_hbm.at[jobs[0][0]], idx0_v)
    g0 = pltpu.async_copy(ys_hbm.at[idx0_v], rows0_v, sg0)
    pltpu.sync_copy(pos_hbm.at[jobs[1][0]], idx1_v)
    g1 = pltpu.async_copy(ys_hbm.at[idx1_v], rows1_v, sg1)
    g0.wait()
    w0 = pltpu.async_copy(rows0_v, jobs[0][1].at[jobs[0][2]], sw0)
    g1.wait()
    w1 = pltpu.async_copy(rows1_v, jobs[1][1].at[jobs[1][2]], sw1)
    w0.wait()
    pltpu.sync_copy(pos_hbm.at[jobs[2][0]], idx0_v)
    g2 = pltpu.async_copy(ys_hbm.at[idx0_v], rows0_v, sg0)
    w1.wait()
    pltpu.sync_copy(pos_hbm.at[jobs[3][0]], idx1_v)
    g3 = pltpu.async_copy(ys_hbm.at[idx1_v], rows1_v, sg1)
    g2.wait()
    w2 = pltpu.async_copy(rows0_v, jobs[2][1].at[jobs[2][2]], sw0)
    g3.wait()
    w3 = pltpu.async_copy(rows1_v, jobs[3][1].at[jobs[3][2]], sw1)
    w2.wait()
    w3.wait()


def _combine(ys, pos2):
    mesh = plsc.VectorSubcoreMesh(core_axis_name="c", subcore_axis_name="s", num_cores=NC, num_subcores=NS)
    f = pl.kernel(
        _combine_body,
        out_type=[
            jax.ShapeDtypeStruct((T, D // 2), jnp.int32),
            jax.ShapeDtypeStruct((T, D // 2), jnp.int32),
        ],
        mesh=mesh,
        scratch_types=[
            pltpu.VMEM((128,), jnp.int32),
            pltpu.VMEM((128,), jnp.int32),
            pltpu.VMEM((128, D // 2), jnp.int32),
            pltpu.VMEM((128, D // 2), jnp.int32),
            pltpu.SemaphoreType.DMA,
            pltpu.SemaphoreType.DMA,
            pltpu.SemaphoreType.DMA,
            pltpu.SemaphoreType.DMA,
        ],
        compiler_params=pltpu.CompilerParams(needs_layout_passes=False),
    )
    return f(ys, pos2)


def _sum_body(a_ref, b_ref, w_ref, o_ref):
    a = _unpack_bf16(a_ref[...]).astype(jnp.float32)
    b = _unpack_bf16(b_ref[...]).astype(jnp.float32)
    o_ref[...] = w_ref[0, :][:, None] * a + w_ref[1, :][:, None] * b


def _sum(ya, yb, wgt):
    return pl.pallas_call(
        _sum_body,
        grid=(T // RBLK,),
        in_specs=[
            pl.BlockSpec((RBLK, D // 2), lambda b: (b, 0)),
            pl.BlockSpec((RBLK, D // 2), lambda b: (b, 0)),
            pl.BlockSpec((2, RBLK), lambda b: (0, b)),
        ],
        out_specs=pl.BlockSpec((RBLK, D), lambda b: (b, 0)),
        out_shape=jax.ShapeDtypeStruct((T, D), jnp.float32),
        compiler_params=pltpu.CompilerParams(
            dimension_semantics=("parallel",),
        ),
    )(ya, yb, wgt)


@jax.jit
def kernel(x, Wr, We, be):
    Bx, Sx, Dx = x.shape
    xf = x.reshape(T, D)

    eid, wgt, cnt, xp = _router(xf, Wr)
    eid2 = eid.reshape(A // 128, 128)
    cnt2 = cnt.reshape(NW, 128)

    pos2, xs, bexp = _dispatch(eid2, cnt2, xp)

    We_bf = We.astype(jnp.bfloat16)
    be3 = be.reshape(E, 1, D)
    ys = _experts(bexp, xs, We_bf, be3)

    ya, yb = _combine(ys, pos2)
    out = _sum(ya, yb, wgt)
    return out.reshape(Bx, Sx, Dx)


# speedup vs baseline: 1.5817x; 1.0020x over previous
import jax
import jax.numpy as jnp
from jax import lax
from jax.experimental import pallas as pl
from jax.experimental.pallas import tpu as pltpu
from jax.experimental.pallas import tpu_sc as plsc

E = 8
D = 768
T = 8192
A = 2 * T
M = 512
_MSHIFT = 9
NSLOT = A + E * M
NB = NSLOT // M
NBP = 48

NC = 2
NS = 16
NW = NC * NS
APW = A // NW
TPW = T // NW

RBLK = 512
_NEG_INF = -1e30


def _pack_bf16(h):
    u = lax.bitcast_convert_type(h, jnp.uint32)
    r = (u + 0x7FFF + ((u >> 16) & 1)) >> 16
    lo = r[:, : D // 2] & 0xFFFF
    hi = r[:, D // 2 :] & 0xFFFF
    return lax.bitcast_convert_type(lo | (hi << 16), jnp.int32)


def _unpack_bf16(p):
    v = lax.bitcast_convert_type(p, jnp.uint32)
    lo = (v & 0xFFFF).astype(jnp.uint16)
    hi = (v >> 16).astype(jnp.uint16)
    return jnp.concatenate(
        [
            lax.bitcast_convert_type(lo, jnp.bfloat16),
            lax.bitcast_convert_type(hi, jnp.bfloat16),
        ],
        axis=1,
    )


def _router_body(x_ref, wr_ref, eid_ref, wgt_ref, cnt_ref, xp_ref):
    xb = x_ref[...]
    xp_ref[...] = _pack_bf16(xb)
    logits = jnp.dot(xb, wr_ref[...])
    m1 = jnp.max(logits, axis=-1)
    i1 = jnp.argmax(logits, axis=-1)
    cols = lax.broadcasted_iota(jnp.int32, logits.shape, 1)
    masked = jnp.where(cols == i1[:, None], _NEG_INF, logits)
    m2 = jnp.max(masked, axis=-1)
    i2 = jnp.argmax(masked, axis=-1)
    w1 = 1.0 / (1.0 + jnp.exp(m2 - m1))
    eid_ref[0:1, :] = i1[None, :]
    eid_ref[1:2, :] = i2[None, :]
    wgt_ref[0:1, :] = w1[None, :]
    wgt_ref[1:2, :] = (1.0 - w1)[None, :]
    lanes = lax.broadcasted_iota(jnp.int32, (RBLK, 128), 1)
    cnt_ref[0, 0] = jnp.sum((i1[:, None] == lanes).astype(jnp.int32), axis=0)[None, :]
    cnt_ref[1, 0] = jnp.sum((i2[:, None] == lanes).astype(jnp.int32), axis=0)[None, :]


def _router(xf, Wr):
    nb = T // RBLK
    return pl.pallas_call(
        _router_body,
        grid=(nb,),
        in_specs=[
            pl.BlockSpec((RBLK, D), lambda b: (b, 0)),
            pl.BlockSpec((D, E), lambda b: (0, 0)),
        ],
        out_specs=[
            pl.BlockSpec((2, RBLK), lambda b: (0, b)),
            pl.BlockSpec((2, RBLK), lambda b: (0, b)),
            pl.BlockSpec((2, 1, 1, 128), lambda b: (0, b, 0, 0)),
            pl.BlockSpec((RBLK, D // 2), lambda b: (b, 0)),
        ],
        out_shape=[
            jax.ShapeDtypeStruct((2, T), jnp.int32),
            jax.ShapeDtypeStruct((2, T), jnp.float32),
            jax.ShapeDtypeStruct((2, nb, 1, 128), jnp.int32),
            jax.ShapeDtypeStruct((T, D // 2), jnp.int32),
        ],
        compiler_params=pltpu.CompilerParams(
            dimension_semantics=("parallel",),
        ),
    )(xf, Wr)


def _iota16():
    return lax.iota(jnp.int32, 16)


def _lane(v, e):
    return lax.squeeze(lax.slice(v, (e,), (e + 1,)), (0,))


def _dispatch_body(eid_hbm, cnt_hbm, xp_hbm,
                  pos_hbm, xs_hbm, bexp_hbm,
                  eid_v, slots_v, tok_v, cnt_v, rows0_v, rows1_v, bexp_v,
                  sg0, sg1, ss0, ss1):
    cid = lax.axis_index("c")
    sid = lax.axis_index("s")
    wid = sid * NC + cid

    pltpu.sync_copy(eid_hbm.at[pl.ds(wid * 4, 4)], eid_v)
    pltpu.sync_copy(cnt_hbm, cnt_v)

    for j in range(APW // 16):
        r, o = j // 8, (j % 8) * 16
        flat = wid * APW + j * 16
        tok_v[r, pl.ds(o, 16)] = (flat + _iota16()) & (T - 1)
    g0 = pltpu.async_copy(xp_hbm.at[tok_v.at[0]], rows0_v, sg0)
    g1 = pltpu.async_copy(xp_hbm.at[tok_v.at[1]], rows1_v, sg1)

    zero = jnp.zeros((16,), jnp.int32)
    tot = zero
    pre = zero
    wid_v = jnp.full((16,), wid, jnp.int32)
    for r in range(NW):
        row = cnt_v[r, pl.ds(0, 16)]
        tot = tot + row
        r_v = jnp.full((16,), r, jnp.int32)
        pre = pre + jnp.where(r_v < wid_v, row, zero)
    aligned = ((tot + (M - 1)) >> _MSHIFT) << _MSHIFT
    base = plsc.cumsum(aligned) - aligned
    start = base + pre

    starts = [_lane(start, e) for e in range(E)]
    for j in range(APW // 16):
        r, o = j // 8, (j % 8) * 16
        v = eid_v[r, pl.ds(o, 16)]
        ms = [v == e for e in range(E)]
        cs = [plsc.cumsum(m.astype(jnp.int32)) for m in ms]
        pcs = [plsc.all_reduce_population_count(m) for m in ms]
        slot = zero
        for e in range(E):
            slot = jnp.where(ms[e], starts[e] + cs[e] - 1, slot)
            starts[e] = starts[e] + pcs[e]
        slots_v[r, pl.ds(o, 16)] = slot

    pltpu.sync_copy(slots_v, pos_hbm.at[pl.ds(wid * 4, 4)])

    g0.wait()
    s0 = pltpu.async_copy(rows0_v, xs_hbm.at[slots_v.at[0]], ss0)
    g1.wait()
    s1 = pltpu.async_copy(rows1_v, xs_hbm.at[slots_v.at[1]], ss1)
    s0.wait()
    g2 = pltpu.async_copy(xp_hbm.at[tok_v.at[2]], rows0_v, sg0)
    s1.wait()
    g3 = pltpu.async_copy(xp_hbm.at[tok_v.at[3]], rows1_v, sg1)
    g2.wait()
    s2 = pltpu.async_copy(rows0_v, xs_hbm.at[slots_v.at[2]], ss0)
    g3.wait()
    s3 = pltpu.async_copy(rows1_v, xs_hbm.at[slots_v.at[3]], ss1)
    s2.wait()
    s3.wait()

    @pl.when(wid == 0)
    def _():
        for q in range(NBP // 16):
            bstart = (_iota16() + q * 16) * M
            acc = jnp.zeros((16,), jnp.int32)
            for e in range(1, E):
                acc = acc + (bstart >= _lane(base, e)).astype(jnp.int32)
            bexp_v[pl.ds(q * 16, 16)] = acc
        pltpu.sync_copy(bexp_v, bexp_hbm)


def _dispatch(eid2, cnt2, xp):
    mesh = plsc.VectorSubcoreMesh(core_axis_name="c", subcore_axis_name="s", num_cores=NC, num_subcores=NS)
    f = pl.kernel(
        _dispatch_body,
        out_type=[
            jax.ShapeDtypeStruct((A // 128, 128), jnp.int32),
            jax.ShapeDtypeStruct((NSLOT, D // 2), jnp.int32),
            jax.ShapeDtypeStruct((NBP,), jnp.int32),
        ],
        mesh=mesh,
        scratch_types=[
            pltpu.VMEM((4, 128), jnp.int32),
            pltpu.VMEM((4, 128), jnp.int32),
            pltpu.VMEM((4, 128), jnp.int32),
            pltpu.VMEM((NW, 128), jnp.int32),
            pltpu.VMEM((128, D // 2), jnp.int32),
            pltpu.VMEM((128, D // 2), jnp.int32),
            pltpu.VMEM((NBP,), jnp.int32),
            pltpu.SemaphoreType.DMA,
            pltpu.SemaphoreType.DMA,
            pltpu.SemaphoreType.DMA,
            pltpu.SemaphoreType.DMA,
        ],
        compiler_params=pltpu.CompilerParams(needs_layout_passes=False),
    )
    return f(eid2, cnt2, xp)


def _expert_body(bexp_ref, xs_ref, we_ref, be_ref, ys_ref):
    xb = _unpack_bf16(xs_ref[...])
    h = jnp.dot(xb, we_ref[0], preferred_element_type=jnp.float32) + be_ref[0]
    h = 0.5 * h * (1.0 + lax.erf(h * 0.7071067811865476))
    ys_ref[...] = _pack_bf16(h)


def _experts(bexp, xs, We_bf, be3):
    grid_spec = pltpu.PrefetchScalarGridSpec(
        num_scalar_prefetch=1,
        grid=(NB,),
        in_specs=[
            pl.BlockSpec((M, D // 2), lambda b, bm: (b, 0)),
            pl.BlockSpec((1, D, D), lambda b, bm: (bm[b], 0, 0)),
            pl.BlockSpec((1, 1, D), lambda b, bm: (bm[b], 0, 0)),
        ],
        out_specs=pl.BlockSpec((M, D // 2), lambda b, bm: (b, 0)),
    )
    return pl.pallas_call(
        _expert_body,
        grid_spec=grid_spec,
        out_shape=jax.ShapeDtypeStruct((NSLOT, D // 2), jnp.int32),
        compiler_params=pltpu.CompilerParams(
            dimension_semantics=("arbitrary",),
        ),
    )(bexp, xs, We_bf, be3)


def _combine_body(ys_hbm, pos_hbm, ya_hbm, yb_hbm,
                  idx0_v, idx1_v, rows0_v, rows1_v, sg0, sg1, sw0, sw1):
    cid = lax.axis_index("c")
    sid = lax.axis_index("s")
    wid = sid * NC + cid

    jobs = []
    for h in range(2):
        dst = pl.ds(wid * TPW + h * 128, 128)
        jobs.append((wid * 2 + h, ya_hbm, dst))
        jobs.append((64 + wid * 2 + h, yb_hbm, dst))

    pltpu.sync_copy(pos_hbm.at[jobs[0][0]], idx0_v)
    g0 = pltpu.async_copy(ys_hbm.at[idx0_v], rows0_v, sg0)
    pltpu.sync_copy(pos_hbm.at[jobs[1][0]], idx1_v)
    g1 = pltpu.async_copy(ys_hbm.at[idx1_v], rows1_v, sg1)
    g0.wait()
    w0 = pltpu.async_copy(rows0_v, jobs[0][1].at[jobs[0][2]], sw0)
    g1.wait()
    w1 = pltpu.async_copy(rows1_v, jobs[1][1].at[jobs[1][2]], sw1)
    w0.wait()
    pltpu.sync_copy(pos_hbm.at[jobs[2][0]], idx0_v)
    g2 = pltpu.async_copy(ys_hbm.at[idx0_v], rows0_v, sg0)
    w1.wait()
    pltpu.sync_copy(pos_hbm.at[jobs[3][0]], idx1_v)
    g3 = pltpu.async_copy(ys_hbm.at[idx1_v], rows1_v, sg1)
    g2.wait()
    w2 = pltpu.async_copy(rows0_v, jobs[2][1].at[jobs[2][2]], sw0)
    g3.wait()
    w3 = pltpu.async_copy(rows1_v, jobs[3][1].at[jobs[3][2]], sw1)
    w2.wait()
    w3.wait()


def _combine(ys, pos2):
    mesh = plsc.VectorSubcoreMesh(core_axis_name="c", subcore_axis_name="s", num_cores=NC, num_subcores=NS)
    f = pl.kernel(
        _combine_body,
        out_type=[
            jax.ShapeDtypeStruct((T, D // 2), jnp.int32),
            jax.ShapeDtypeStruct((T, D // 2), jnp.int32),
        ],
        mesh=mesh,
        scratch_types=[
            pltpu.VMEM((128,), jnp.int32),
            pltpu.VMEM((128,), jnp.int32),
            pltpu.VMEM((128, D // 2), jnp.int32),
            pltpu.VMEM((128, D // 2), jnp.int32),
            pltpu.SemaphoreType.DMA,
            pltpu.SemaphoreType.DMA,
            pltpu.SemaphoreType.DMA,
            pltpu.SemaphoreType.DMA,
        ],
        compiler_params=pltpu.CompilerParams(needs_layout_passes=False),
    )
    return f(ys, pos2)


def _sum_body(a_ref, b_ref, w_ref, o_ref):
    a = _unpack_bf16(a_ref[...]).astype(jnp.float32)
    b = _unpack_bf16(b_ref[...]).astype(jnp.float32)
    o_ref[...] = w_ref[0, :][:, None] * a + w_ref[1, :][:, None] * b


def _sum(ya, yb, wgt):
    return pl.pallas_call(
        _sum_body,
        grid=(T // RBLK,),
        in_specs=[
            pl.BlockSpec((RBLK, D // 2), lambda b: (b, 0)),
            pl.BlockSpec((RBLK, D // 2), lambda b: (b, 0)),
            pl.BlockSpec((2, RBLK), lambda b: (0, b)),
        ],
        out_specs=pl.BlockSpec((RBLK, D), lambda b: (b, 0)),
        out_shape=jax.ShapeDtypeStruct((T, D), jnp.float32),
        compiler_params=pltpu.CompilerParams(
            dimension_semantics=("parallel",),
        ),
    )(ya, yb, wgt)


@jax.jit
def kernel(x, Wr, We, be):
    Bx, Sx, Dx = x.shape
    xf = x.reshape(T, D)

    eid, wgt, cnt, xp = _router(xf, Wr)
    eid2 = eid.reshape(A // 128, 128)
    cnt2 = cnt.reshape(NW, 128)

    pos2, xs, bexp = _dispatch(eid2, cnt2, xp)

    We_bf = We.astype(jnp.bfloat16)
    be3 = be.reshape(E, 1, D)
    ys = _experts(bexp, xs, We_bf, be3)

    ya, yb = _combine(ys, pos2)
    out = _sum(ya, yb, wgt)
    return out.reshape(Bx, Sx, Dx)
